# baseline (speedup 1.0000x reference)
import functools

import jax
import jax.numpy as jnp
from jax.experimental import pallas as pl
import jax.experimental.pallas.tpu as pltpu

REG_MAX = 16
TOPK = 13
W_MATCH = 0.5
W_IOU = 7.5
W_DFL = 1.5
W_CONTRAST = 1.0

_R = 160
_L = 128
_NP = _R * _L


def _loss_kernel(ps_ref, pb_ref, bd_ref, an_ref, st_ref, gt_ref, pv_ref,
                 img_ref, out_ref, metric, ovl, tx1, ty1, tx2, ty2,
                 alsc, iosc, cvsc, cisc):
    G = gt_ref.shape[1]
    nreal_f = img_ref[0, 1]
    nreal_i = nreal_f.astype(jnp.int32)

    ax = an_ref[0]
    ay = an_ref[1]
    idx = (jax.lax.broadcasted_iota(jnp.int32, (_R, _L), 0) * _L
           + jax.lax.broadcasted_iota(jnp.int32, (_R, _L), 1))
    valid = idx < nreal_i

    px1 = pb_ref[0, 0]
    py1 = pb_ref[0, 1]
    px2 = pb_ref[0, 2]
    py2 = pb_ref[0, 3]
    parea = (px2 - px1) * (py2 - py1)
    sig = jax.nn.sigmoid(ps_ref[0])

    metric[...] = jnp.full((_R, _L), -1.0, jnp.float32)
    ovl[...] = jnp.zeros((_R, _L), jnp.float32)
    tx1[...] = jnp.zeros((_R, _L), jnp.float32)
    ty1[...] = jnp.zeros((_R, _L), jnp.float32)
    tx2[...] = jnp.zeros((_R, _L), jnp.float32)
    ty2[...] = jnp.zeros((_R, _L), jnp.float32)

    riota = jax.lax.broadcasted_iota(jnp.int32, (_R, _L), 0)
    liota1 = jax.lax.broadcasted_iota(jnp.int32, (1, _L), 1)
    lmod16 = liota1 % 16
    BIGI = jnp.int32(2**30)

    def _seg16_all(v, op):
        for s in (1, 2, 4, 8):
            a = pltpu.roll(v, s, axis=1)
            b = pltpu.roll(v, s + _L - 16, axis=1)
            v = op(v, jnp.where(lmod16 >= s, a, b))
        return v

    def _bcast_group(v, j):
        if j:
            v = pltpu.roll(v, _L - 16 * j, axis=1)
        for s in (16, 32, 64):
            v = jnp.where(liota1 % (2 * s) >= s, pltpu.roll(v, s, axis=1), v)
        return v

    def _phase1(g):
        gx1 = gt_ref[0, g, 0]
        gy1 = gt_ref[0, g, 1]
        gx2 = gt_ref[0, g, 2]
        gy2 = gt_ref[0, g, 3]
        ix1 = jnp.maximum(px1, gx1)
        iy1 = jnp.maximum(py1, gy1)
        ix2 = jnp.minimum(px2, gx2)
        iy2 = jnp.minimum(py2, gy2)
        inter = jnp.maximum(ix2 - ix1, 0.0) * jnp.maximum(iy2 - iy1, 0.0)
        garea = (gx2 - gx1) * (gy2 - gy1)
        iou = inter / (parea + garea - inter + 1e-7)
        inside = ((ax >= gx1) & (ax <= gx2) & (ay >= gy1) & (ay <= gy2)
                  & valid)
        iou2 = iou * iou
        align = jnp.where(inside, sig * (iou2 * iou2 * iou2), -3.0)

        work = align
        cand_v = []
        cand_i = []
        for _ in range(TOPK):
            m = jnp.max(work, axis=0, keepdims=True)
            rs = jnp.min(jnp.where(work == m, riota, jnp.int32(_R)),
                         axis=0, keepdims=True)
            cand_v.append(m)
            cand_i.append(rs * _L + liota1)
            work = jnp.where(riota == rs, -4.0, work)
        cand_v.extend([jnp.full((1, _L), -4.0, jnp.float32)] * 3)
        cand_i.extend([jnp.full((1, _L), BIGI, jnp.int32)] * 3)
        return (align, iou, jnp.concatenate(cand_v, axis=0),
                jnp.concatenate(cand_i, axis=0))

    def grp_step(grp, _):
        def pair_step(p, _):
            for jj in range(2):
                j = p * 2 + jj
                al, io, cv, ci = _phase1(grp * 8 + j)
                alsc[pl.ds(j, 1)] = al[None]
                iosc[pl.ds(j, 1)] = io[None]
                cvsc[pl.ds(j * 16, 16)] = cv
                cisc[pl.ds(j * 16, 16)] = ci
            return 0

        jax.lax.fori_loop(0, 4, pair_step, 0)
        Wt = jnp.transpose(cvsc[...])
        It = jnp.transpose(cisc[...])
        mb = None
        ib = None
        for _ in range(TOPK):
            mb = _seg16_all(jnp.max(Wt, axis=0, keepdims=True), jnp.maximum)
            i1 = jnp.min(jnp.where(Wt == mb, It, BIGI), axis=0,
                         keepdims=True)
            ib = _seg16_all(i1, jnp.minimum)
            Wt = jnp.where(It == ib, -4.0, Wt)
        for j in range(8):
            g = grp * 8 + j
            t = _bcast_group(mb, j)
            ti = _bcast_group(ib, j)
            align = alsc[j]
            iou = iosc[j]
            selmask = (align > t) | ((align == t) & (idx <= ti))
            upd = selmask & (align > metric[...])
            metric[...] = jnp.where(upd, align, metric[...])
            ovl[...] = jnp.where(upd, iou, ovl[...])
            tx1[...] = jnp.where(upd, gt_ref[0, g, 0], tx1[...])
            ty1[...] = jnp.where(upd, gt_ref[0, g, 1], ty1[...])
            tx2[...] = jnp.where(upd, gt_ref[0, g, 2], tx2[...])
            ty2[...] = jnp.where(upd, gt_ref[0, g, 3], ty2[...])
        return 0

    jax.lax.fori_loop(0, G // 8, grp_step, 0)

    fg = metric[...] > -0.5
    fgf = jnp.where(fg, 1.0, 0.0)
    npos = jnp.sum(fgf)
    denom = jnp.maximum(npos, 1.0)

    ts = jnp.where(fg, jnp.maximum(ovl[...], 0.1), 0.0)
    x = ps_ref[0]
    bce = (jnp.maximum(x, 0.0) - x * ts
           + jnp.log1p(jnp.exp(-jnp.abs(x))))
    match_b = jnp.sum(jnp.where(valid, bce, 0.0)) / nreal_f

    prob = sig
    pos_score_b = jnp.sum(jnp.where(fg, prob, 0.0))
    neg_score_b = jnp.sum(jnp.where(valid & (~fg), prob, 0.0))

    bx1 = jnp.where(fg, tx1[...], px1)
    by1 = jnp.where(fg, ty1[...], py1)
    bx2 = jnp.where(fg, tx2[...], px2)
    by2 = jnp.where(fg, ty2[...], py2)
    eps = 1e-7
    ix1 = jnp.maximum(px1, bx1)
    iy1 = jnp.maximum(py1, by1)
    ix2 = jnp.minimum(px2, bx2)
    iy2 = jnp.minimum(py2, by2)
    inter = jnp.maximum(ix2 - ix1, 0.0) * jnp.maximum(iy2 - iy1, 0.0)
    a2 = (bx2 - bx1) * (by2 - by1)
    iou = inter / (parea + a2 - inter + eps)
    matched_iou_b = jnp.sum(jnp.where(fg, iou, 0.0))
    cw = jnp.maximum(px2, bx2) - jnp.minimum(px1, bx1)
    ch = jnp.maximum(py2, by2) - jnp.minimum(py1, by1)
    c2 = cw * cw + ch * ch + eps
    rho2 = ((bx1 + bx2 - px1 - px2) ** 2 + (by1 + by2 - py1 - py2) ** 2) / 4.0
    w1 = px2 - px1 + eps
    h1 = py2 - py1 + eps
    w2 = bx2 - bx1 + eps
    h2 = by2 - by1 + eps
    pi2 = 9.869604401089358
    z = (w2 * h1 - w1 * h2) / (h1 * h2 + w1 * w2)
    az = jnp.abs(z)
    big = az > 1.0
    y = jnp.where(big, 1.0 / az, az)
    t = y * y
    p = jnp.float32(0.0028340642986113477)
    for coef in (-0.01600503050194432, 0.042587607462732255,
                 -0.0749544544309546, 0.10636754098013634,
                 -0.14202570511671397, 0.19992483578497475,
                 -0.3333306678069131, 0.9999999842426359):
        p = p * t + jnp.float32(coef)
    aty = y * p
    dang = jnp.sign(z) * jnp.where(big, 1.5707963267948966 - aty, aty)
    v = (4.0 / pi2) * dang * dang
    alpha = v / (v - iou + 1.0 + eps)
    ciou = iou - (rho2 / c2 + v * alpha)
    iou_b = jnp.sum(jnp.where(fg, 1.0 - ciou, 0.0)) / denom

    stv = st_ref[0]
    dsum = jnp.float32(0.0)
    tgt0 = (ax - bx1) / stv
    tgt1 = (ay - by1) / stv
    tgt2 = (bx2 - ax) / stv
    tgt3 = (by2 - ay) / stv
    for c, tgt in enumerate((tgt0, tgt1, tgt2, tgt3)):
        tgt = jnp.clip(tgt, 0.0, REG_MAX - 1 - 0.01)
        tl = tgt.astype(jnp.int32)
        tr = jnp.minimum(tl + 1, REG_MAX - 1)
        wl = tr.astype(jnp.float32) - tgt
        wr = 1.0 - wl
        lg = bd_ref[0, c * REG_MAX:(c + 1) * REG_MAX]
        m16 = jnp.max(lg, axis=0)
        s = jnp.sum(jnp.exp(lg - m16[None]), axis=0)
        lse = m16 + jnp.log(s)
        j3 = jax.lax.broadcasted_iota(jnp.int32, (REG_MAX, _R, _L), 0)
        l_tl = jnp.sum(jnp.where(tl[None] == j3, lg, 0.0), axis=0)
        l_tr = jnp.sum(jnp.where(tr[None] == j3, lg, 0.0), axis=0)
        dfl_c = (lse - l_tl) * wl + (lse - l_tr) * wr
        dsum = dsum + jnp.sum(jnp.where(fg, dfl_c, 0.0))
    dfl_b = dsum / (denom * 4.0)

    img = img_ref[0, 0].astype(jnp.float32)
    cx = (bx1 + bx2) * 0.5 / img
    cy = (by1 + by2) * 0.5 / img
    nrm = jnp.maximum(jnp.sqrt(cx * cx + cy * cy), 1e-12)
    pvx = pv_ref[0, 0, 0]
    pvy = pv_ref[0, 0, 1]
    pn = jnp.maximum(jnp.sqrt(pvx * pvx + pvy * pvy), 1e-12)
    contrast = 1.0 - (cx * (pvx / pn) + cy * (pvy / pn)) / nrm
    contrast_b = jnp.sum(jnp.where(fg, contrast, 0.0)) / denom

    out_ref[0, 0,0] = match_b
    out_ref[0, 0,1] = iou_b
    out_ref[0, 0,2] = dfl_b
    out_ref[0, 0,3] = contrast_b
    out_ref[0, 0,4] = npos
    out_ref[0, 0,5] = nreal_f - npos
    out_ref[0, 0,6] = pos_score_b
    out_ref[0, 0,7] = neg_score_b
    out_ref[0, 0,8] = matched_iou_b


def kernel(pred_boxes, pred_scores, anchor_points, stride_tensor,
           box_distribution, prompt_embedding, gt_boxes, image_size):
    B, N = pred_scores.shape
    G = gt_boxes.shape[1]
    pad = _NP - N

    ps = jnp.pad(pred_scores, ((0, 0), (0, pad))).reshape(B, _R, _L)
    pb = jnp.pad(jnp.transpose(pred_boxes, (0, 2, 1)),
                 ((0, 0), (0, 0), (0, pad))).reshape(B, 4, _R, _L)
    bd = jnp.pad(jnp.transpose(box_distribution, (0, 2, 3, 1)),
                 ((0, 0), (0, 0), (0, 0), (0, pad))
                 ).reshape(B, 4 * REG_MAX, _R, _L)
    an = jnp.pad(jnp.transpose(anchor_points, (1, 0)),
                 ((0, 0), (0, pad))).reshape(2, _R, _L)
    st = jnp.pad(stride_tensor, ((0, pad),),
                 constant_values=1.0).reshape(1, _R, _L)
    pv = prompt_embedding[:, :2].reshape(B, 1, 2)
    img = jnp.maximum(jnp.asarray(image_size, jnp.float32), 1.0)
    imgn = jnp.stack([img, jnp.float32(N)]).reshape(1, 2)

    out = pl.pallas_call(
        _loss_kernel,
        grid=(B,),
        in_specs=[
            pl.BlockSpec((1, _R, _L), lambda b: (b, 0, 0)),
            pl.BlockSpec((1, 4, _R, _L), lambda b: (b, 0, 0, 0)),
            pl.BlockSpec((1, 4 * REG_MAX, _R, _L), lambda b: (b, 0, 0, 0)),
            pl.BlockSpec((2, _R, _L), lambda b: (0, 0, 0)),
            pl.BlockSpec((1, _R, _L), lambda b: (0, 0, 0)),
            pl.BlockSpec((1, G, 4), lambda b: (b, 0, 0),
                         memory_space=pltpu.SMEM),
            pl.BlockSpec((1, 1, 2), lambda b: (b, 0, 0),
                         memory_space=pltpu.SMEM),
            pl.BlockSpec((1, 2), lambda b: (0, 0), memory_space=pltpu.SMEM),
        ],
        out_specs=pl.BlockSpec((1, 1, 16), lambda b: (b, 0, 0),
                               memory_space=pltpu.SMEM),
        out_shape=jax.ShapeDtypeStruct((B, 1, 16), jnp.float32),
        scratch_shapes=(
            [pltpu.VMEM((_R, _L), jnp.float32) for _ in range(6)]
            + [pltpu.VMEM((8, _R, _L), jnp.float32) for _ in range(2)]
            + [pltpu.VMEM((_L, _L), jnp.float32),
               pltpu.VMEM((_L, _L), jnp.int32)]),
        compiler_params=pltpu.CompilerParams(
            dimension_semantics=("parallel",)),
    )(ps, pb, bd, an, st, gt_boxes, pv, imgn)

    out = out[:, 0]
    match = out[:, 0]
    iou_l = out[:, 1]
    dfl_l = out[:, 2]
    contrast = out[:, 3]
    npos = out[:, 4]
    nneg = out[:, 5]
    pos_score = out[:, 6]
    neg_score = out[:, 7]
    matched_iou = out[:, 8]

    nb = float(B)
    total_match = jnp.sum(match)
    total_iou = jnp.sum(iou_l)
    total_dfl = jnp.sum(dfl_l)
    total_contrast = jnp.sum(contrast)
    total_pos = jnp.sum(npos)
    total_neg = jnp.sum(nneg)
    loss = (W_MATCH * total_match + W_IOU * total_iou + W_DFL * total_dfl
            + W_CONTRAST * total_contrast) / nb
    mean_pos_score = jnp.sum(pos_score) / jnp.maximum(total_pos, 1.0)
    mean_neg_score = jnp.sum(neg_score) / jnp.maximum(total_neg, 1.0)
    mean_matched_iou = jnp.sum(matched_iou) / jnp.maximum(total_pos, 1.0)
    return (loss, total_match / nb, total_iou / nb, total_dfl / nb,
            total_contrast / nb, total_pos, total_neg, mean_pos_score,
            mean_neg_score, mean_matched_iou)


# speedup vs baseline: 1.0871x; 1.0871x over previous
import functools

import jax
import jax.numpy as jnp
from jax.experimental import pallas as pl
import jax.experimental.pallas.tpu as pltpu

REG_MAX = 16
TOPK = 13
W_MATCH = 0.5
W_IOU = 7.5
W_DFL = 1.5
W_CONTRAST = 1.0

_R = 160
_L = 128
_NP = _R * _L


def _loss_kernel(ps_ref, pb_ref, bd_ref, an_ref, st_ref, gt_ref, pv_ref,
                 img_ref, out_ref, metric, ovl, tx1, ty1, tx2, ty2,
                 alsc, iosc, cvsc, cisc):
    G = gt_ref.shape[1]
    nreal_f = img_ref[0, 1]
    nreal_i = nreal_f.astype(jnp.int32)

    ax = an_ref[0]
    ay = an_ref[1]
    idx = (jax.lax.broadcasted_iota(jnp.int32, (_R, _L), 0) * _L
           + jax.lax.broadcasted_iota(jnp.int32, (_R, _L), 1))
    valid = idx < nreal_i

    px1 = pb_ref[0, 0]
    py1 = pb_ref[0, 1]
    px2 = pb_ref[0, 2]
    py2 = pb_ref[0, 3]
    parea = (px2 - px1) * (py2 - py1)
    sig = jax.nn.sigmoid(ps_ref[0])

    metric[...] = jnp.full((_R, _L), -1.0, jnp.float32)
    ovl[...] = jnp.zeros((_R, _L), jnp.float32)
    tx1[...] = jnp.zeros((_R, _L), jnp.float32)
    ty1[...] = jnp.zeros((_R, _L), jnp.float32)
    tx2[...] = jnp.zeros((_R, _L), jnp.float32)
    ty2[...] = jnp.zeros((_R, _L), jnp.float32)

    riota = jax.lax.broadcasted_iota(jnp.int32, (_R, _L), 0)
    liota1 = jax.lax.broadcasted_iota(jnp.int32, (1, _L), 1)
    lmod16 = liota1 % 16
    BIGI = jnp.int32(2**30)

    def _seg16_all(v, op):
        for s in (1, 2, 4, 8):
            a = pltpu.roll(v, s, axis=1)
            b = pltpu.roll(v, s + _L - 16, axis=1)
            v = op(v, jnp.where(lmod16 >= s, a, b))
        return v

    def _bcast_group(v, j):
        if j:
            v = pltpu.roll(v, _L - 16 * j, axis=1)
        for s in (16, 32, 64):
            v = jnp.where(liota1 % (2 * s) >= s, pltpu.roll(v, s, axis=1), v)
        return v

    rowkey = 255 - riota
    MININT = jnp.int32(-2**31)

    def _key_from_align(align):
        bits = jax.lax.bitcast_convert_type(align, jnp.int32)
        return jnp.where(align >= 0.0, (bits & jnp.int32(-256)) | rowkey,
                         jnp.int32(-1))

    def _phase1(g):
        gx1 = gt_ref[0, g, 0]
        gy1 = gt_ref[0, g, 1]
        gx2 = gt_ref[0, g, 2]
        gy2 = gt_ref[0, g, 3]
        ix1 = jnp.maximum(px1, gx1)
        iy1 = jnp.maximum(py1, gy1)
        ix2 = jnp.minimum(px2, gx2)
        iy2 = jnp.minimum(py2, gy2)
        inter = jnp.maximum(ix2 - ix1, 0.0) * jnp.maximum(iy2 - iy1, 0.0)
        garea = (gx2 - gx1) * (gy2 - gy1)
        iou = inter / (parea + garea - inter + 1e-7)
        inside = ((ax >= gx1) & (ax <= gx2) & (ay >= gy1) & (ay <= gy2)
                  & valid)
        iou2 = iou * iou
        align = jnp.where(inside, sig * (iou2 * iou2 * iou2), -3.0)

        work = _key_from_align(align)
        cand_v = []
        cand_i = []
        for _ in range(TOPK):
            m = jnp.max(work, axis=0, keepdims=True)
            cand_v.append(m)
            cand_i.append((255 - (m & 255)) * _L + liota1)
            work = jnp.where(work == m, MININT, work)
        cand_v.extend([jnp.full((1, _L), MININT + 1, jnp.int32)] * 3)
        cand_i.extend([jnp.full((1, _L), BIGI, jnp.int32)] * 3)
        return (align, iou, jnp.concatenate(cand_v, axis=0),
                jnp.concatenate(cand_i, axis=0))

    def grp_step(grp, _):
        for j in range(8):
            al, io, cv, ci = _phase1(grp * 8 + j)
            alsc[j] = al
            iosc[j] = io
            cvsc[j * 16:(j + 1) * 16] = cv
            cisc[j * 16:(j + 1) * 16] = ci
        Wt = jnp.transpose(cvsc[...])
        It = jnp.transpose(cisc[...])
        mb = None
        ib = None
        for _ in range(TOPK):
            mb = _seg16_all(jnp.max(Wt, axis=0, keepdims=True), jnp.maximum)
            i1 = jnp.min(jnp.where(Wt == mb, It, BIGI), axis=0,
                         keepdims=True)
            ib = _seg16_all(i1, jnp.minimum)
            Wt = jnp.where(It == ib, MININT, Wt)
        for j in range(8):
            g = grp * 8 + j
            t = _bcast_group(mb, j)
            ti = _bcast_group(ib, j)
            align = alsc[j]
            iou = iosc[j]
            key = _key_from_align(align)
            selmask = (key > t) | ((key == t) & (idx <= ti))
            upd = selmask & (align > metric[...])
            metric[...] = jnp.where(upd, align, metric[...])
            ovl[...] = jnp.where(upd, iou, ovl[...])
            tx1[...] = jnp.where(upd, gt_ref[0, g, 0], tx1[...])
            ty1[...] = jnp.where(upd, gt_ref[0, g, 1], ty1[...])
            tx2[...] = jnp.where(upd, gt_ref[0, g, 2], tx2[...])
            ty2[...] = jnp.where(upd, gt_ref[0, g, 3], ty2[...])
        return 0

    jax.lax.fori_loop(0, G // 8, grp_step, 0)

    fg = metric[...] > -0.5
    fgf = jnp.where(fg, 1.0, 0.0)
    npos = jnp.sum(fgf)
    denom = jnp.maximum(npos, 1.0)

    ts = jnp.where(fg, jnp.maximum(ovl[...], 0.1), 0.0)
    x = ps_ref[0]
    bce = (jnp.maximum(x, 0.0) - x * ts
           + jnp.log1p(jnp.exp(-jnp.abs(x))))
    match_b = jnp.sum(jnp.where(valid, bce, 0.0)) / nreal_f

    prob = sig
    pos_score_b = jnp.sum(jnp.where(fg, prob, 0.0))
    neg_score_b = jnp.sum(jnp.where(valid & (~fg), prob, 0.0))

    bx1 = jnp.where(fg, tx1[...], px1)
    by1 = jnp.where(fg, ty1[...], py1)
    bx2 = jnp.where(fg, tx2[...], px2)
    by2 = jnp.where(fg, ty2[...], py2)
    eps = 1e-7
    ix1 = jnp.maximum(px1, bx1)
    iy1 = jnp.maximum(py1, by1)
    ix2 = jnp.minimum(px2, bx2)
    iy2 = jnp.minimum(py2, by2)
    inter = jnp.maximum(ix2 - ix1, 0.0) * jnp.maximum(iy2 - iy1, 0.0)
    a2 = (bx2 - bx1) * (by2 - by1)
    iou = inter / (parea + a2 - inter + eps)
    matched_iou_b = jnp.sum(jnp.where(fg, iou, 0.0))
    cw = jnp.maximum(px2, bx2) - jnp.minimum(px1, bx1)
    ch = jnp.maximum(py2, by2) - jnp.minimum(py1, by1)
    c2 = cw * cw + ch * ch + eps
    rho2 = ((bx1 + bx2 - px1 - px2) ** 2 + (by1 + by2 - py1 - py2) ** 2) / 4.0
    w1 = px2 - px1 + eps
    h1 = py2 - py1 + eps
    w2 = bx2 - bx1 + eps
    h2 = by2 - by1 + eps
    pi2 = 9.869604401089358
    z = (w2 * h1 - w1 * h2) / (h1 * h2 + w1 * w2)
    az = jnp.abs(z)
    big = az > 1.0
    y = jnp.where(big, 1.0 / az, az)
    t = y * y
    p = jnp.float32(0.0028340642986113477)
    for coef in (-0.01600503050194432, 0.042587607462732255,
                 -0.0749544544309546, 0.10636754098013634,
                 -0.14202570511671397, 0.19992483578497475,
                 -0.3333306678069131, 0.9999999842426359):
        p = p * t + jnp.float32(coef)
    aty = y * p
    dang = jnp.sign(z) * jnp.where(big, 1.5707963267948966 - aty, aty)
    v = (4.0 / pi2) * dang * dang
    alpha = v / (v - iou + 1.0 + eps)
    ciou = iou - (rho2 / c2 + v * alpha)
    iou_b = jnp.sum(jnp.where(fg, 1.0 - ciou, 0.0)) / denom

    stv = st_ref[0]
    dsum = jnp.float32(0.0)
    tgt0 = (ax - bx1) / stv
    tgt1 = (ay - by1) / stv
    tgt2 = (bx2 - ax) / stv
    tgt3 = (by2 - ay) / stv
    for c, tgt in enumerate((tgt0, tgt1, tgt2, tgt3)):
        tgt = jnp.clip(tgt, 0.0, REG_MAX - 1 - 0.01)
        tl = tgt.astype(jnp.int32)
        tr = jnp.minimum(tl + 1, REG_MAX - 1)
        wl = tr.astype(jnp.float32) - tgt
        wr = 1.0 - wl
        lg = bd_ref[0, c * REG_MAX:(c + 1) * REG_MAX]
        m16 = jnp.max(lg, axis=0)
        s = jnp.sum(jnp.exp(lg - m16[None]), axis=0)
        lse = m16 + jnp.log(s)
        j3 = jax.lax.broadcasted_iota(jnp.int32, (REG_MAX, _R, _L), 0)
        l_tl = jnp.sum(jnp.where(tl[None] == j3, lg, 0.0), axis=0)
        l_tr = jnp.sum(jnp.where(tr[None] == j3, lg, 0.0), axis=0)
        dfl_c = (lse - l_tl) * wl + (lse - l_tr) * wr
        dsum = dsum + jnp.sum(jnp.where(fg, dfl_c, 0.0))
    dfl_b = dsum / (denom * 4.0)

    img = img_ref[0, 0].astype(jnp.float32)
    cx = (bx1 + bx2) * 0.5 / img
    cy = (by1 + by2) * 0.5 / img
    nrm = jnp.maximum(jnp.sqrt(cx * cx + cy * cy), 1e-12)
    pvx = pv_ref[0, 0, 0]
    pvy = pv_ref[0, 0, 1]
    pn = jnp.maximum(jnp.sqrt(pvx * pvx + pvy * pvy), 1e-12)
    contrast = 1.0 - (cx * (pvx / pn) + cy * (pvy / pn)) / nrm
    contrast_b = jnp.sum(jnp.where(fg, contrast, 0.0)) / denom

    out_ref[0, 0,0] = match_b
    out_ref[0, 0,1] = iou_b
    out_ref[0, 0,2] = dfl_b
    out_ref[0, 0,3] = contrast_b
    out_ref[0, 0,4] = npos
    out_ref[0, 0,5] = nreal_f - npos
    out_ref[0, 0,6] = pos_score_b
    out_ref[0, 0,7] = neg_score_b
    out_ref[0, 0,8] = matched_iou_b


def kernel(pred_boxes, pred_scores, anchor_points, stride_tensor,
           box_distribution, prompt_embedding, gt_boxes, image_size):
    B, N = pred_scores.shape
    G = gt_boxes.shape[1]
    pad = _NP - N

    ps = jnp.pad(pred_scores, ((0, 0), (0, pad))).reshape(B, _R, _L)
    pb = jnp.pad(jnp.transpose(pred_boxes, (0, 2, 1)),
                 ((0, 0), (0, 0), (0, pad))).reshape(B, 4, _R, _L)
    bd = jnp.pad(jnp.transpose(box_distribution, (0, 2, 3, 1)),
                 ((0, 0), (0, 0), (0, 0), (0, pad))
                 ).reshape(B, 4 * REG_MAX, _R, _L)
    an = jnp.pad(jnp.transpose(anchor_points, (1, 0)),
                 ((0, 0), (0, pad))).reshape(2, _R, _L)
    st = jnp.pad(stride_tensor, ((0, pad),),
                 constant_values=1.0).reshape(1, _R, _L)
    pv = prompt_embedding[:, :2].reshape(B, 1, 2)
    img = jnp.maximum(jnp.asarray(image_size, jnp.float32), 1.0)
    imgn = jnp.stack([img, jnp.float32(N)]).reshape(1, 2)

    out = pl.pallas_call(
        _loss_kernel,
        grid=(B,),
        in_specs=[
            pl.BlockSpec((1, _R, _L), lambda b: (b, 0, 0)),
            pl.BlockSpec((1, 4, _R, _L), lambda b: (b, 0, 0, 0)),
            pl.BlockSpec((1, 4 * REG_MAX, _R, _L), lambda b: (b, 0, 0, 0)),
            pl.BlockSpec((2, _R, _L), lambda b: (0, 0, 0)),
            pl.BlockSpec((1, _R, _L), lambda b: (0, 0, 0)),
            pl.BlockSpec((1, G, 4), lambda b: (b, 0, 0),
                         memory_space=pltpu.SMEM),
            pl.BlockSpec((1, 1, 2), lambda b: (b, 0, 0),
                         memory_space=pltpu.SMEM),
            pl.BlockSpec((1, 2), lambda b: (0, 0), memory_space=pltpu.SMEM),
        ],
        out_specs=pl.BlockSpec((1, 1, 16), lambda b: (b, 0, 0),
                               memory_space=pltpu.SMEM),
        out_shape=jax.ShapeDtypeStruct((B, 1, 16), jnp.float32),
        scratch_shapes=(
            [pltpu.VMEM((_R, _L), jnp.float32) for _ in range(6)]
            + [pltpu.VMEM((8, _R, _L), jnp.float32) for _ in range(2)]
            + [pltpu.VMEM((_L, _L), jnp.int32),
               pltpu.VMEM((_L, _L), jnp.int32)]),
        compiler_params=pltpu.CompilerParams(
            dimension_semantics=("parallel",)),
    )(ps, pb, bd, an, st, gt_boxes, pv, imgn)

    out = out[:, 0]
    match = out[:, 0]
    iou_l = out[:, 1]
    dfl_l = out[:, 2]
    contrast = out[:, 3]
    npos = out[:, 4]
    nneg = out[:, 5]
    pos_score = out[:, 6]
    neg_score = out[:, 7]
    matched_iou = out[:, 8]

    nb = float(B)
    total_match = jnp.sum(match)
    total_iou = jnp.sum(iou_l)
    total_dfl = jnp.sum(dfl_l)
    total_contrast = jnp.sum(contrast)
    total_pos = jnp.sum(npos)
    total_neg = jnp.sum(nneg)
    loss = (W_MATCH * total_match + W_IOU * total_iou + W_DFL * total_dfl
            + W_CONTRAST * total_contrast) / nb
    mean_pos_score = jnp.sum(pos_score) / jnp.maximum(total_pos, 1.0)
    mean_neg_score = jnp.sum(neg_score) / jnp.maximum(total_neg, 1.0)
    mean_matched_iou = jnp.sum(matched_iou) / jnp.maximum(total_pos, 1.0)
    return (loss, total_match / nb, total_iou / nb, total_dfl / nb,
            total_contrast / nb, total_pos, total_neg, mean_pos_score,
            mean_neg_score, mean_matched_iou)


# speedup vs baseline: 1.1131x; 1.0239x over previous
import functools

import jax
import jax.numpy as jnp
from jax.experimental import pallas as pl
import jax.experimental.pallas.tpu as pltpu

REG_MAX = 16
TOPK = 13
W_MATCH = 0.5
W_IOU = 7.5
W_DFL = 1.5
W_CONTRAST = 1.0

_R = 160
_L = 128
_NP = _R * _L


def _loss_kernel(ps_ref, pb_ref, bd_ref, an_ref, st_ref, gt_ref, pv_ref,
                 img_ref, out_ref, metric, ovl, tx1, ty1, tx2, ty2,
                 alsc, iosc, cvsc, cisc):
    G = gt_ref.shape[1]
    nreal_f = img_ref[0, 1]
    nreal_i = nreal_f.astype(jnp.int32)

    ax = an_ref[0]
    ay = an_ref[1]
    idx = (jax.lax.broadcasted_iota(jnp.int32, (_R, _L), 0) * _L
           + jax.lax.broadcasted_iota(jnp.int32, (_R, _L), 1))
    valid = idx < nreal_i

    px1 = pb_ref[0, 0]
    py1 = pb_ref[0, 1]
    px2 = pb_ref[0, 2]
    py2 = pb_ref[0, 3]
    parea = (px2 - px1) * (py2 - py1)
    sig = jax.nn.sigmoid(ps_ref[0])

    metric[...] = jnp.full((_R, _L), -1.0, jnp.float32)
    ovl[...] = jnp.zeros((_R, _L), jnp.float32)
    tx1[...] = jnp.zeros((_R, _L), jnp.float32)
    ty1[...] = jnp.zeros((_R, _L), jnp.float32)
    tx2[...] = jnp.zeros((_R, _L), jnp.float32)
    ty2[...] = jnp.zeros((_R, _L), jnp.float32)

    riota = jax.lax.broadcasted_iota(jnp.int32, (_R, _L), 0)
    liota1 = jax.lax.broadcasted_iota(jnp.int32, (1, _L), 1)
    lmod16 = liota1 % 16
    BIGI = jnp.int32(2**30)

    def _seg16_all(v, op):
        for s in (1, 2, 4, 8):
            a = pltpu.roll(v, s, axis=1)
            b = pltpu.roll(v, s + _L - 16, axis=1)
            v = op(v, jnp.where(lmod16 >= s, a, b))
        return v

    def _bcast_group(v, j):
        if j:
            v = pltpu.roll(v, _L - 16 * j, axis=1)
        for s in (16, 32, 64):
            v = jnp.where(liota1 % (2 * s) >= s, pltpu.roll(v, s, axis=1), v)
        return v

    rowkey = 255 - riota
    MININT = jnp.int32(-2**31)

    def _key_from_align(align):
        bits = jax.lax.bitcast_convert_type(align, jnp.int32)
        kb = ((bits & jnp.int32(-256)) | rowkey) + jnp.int32(0x40000000)
        return jnp.where(align >= 0.0,
                         jax.lax.bitcast_convert_type(kb, jnp.float32),
                         jnp.float32(1.0))

    def _phase1(g):
        gx1 = gt_ref[0, g, 0]
        gy1 = gt_ref[0, g, 1]
        gx2 = gt_ref[0, g, 2]
        gy2 = gt_ref[0, g, 3]
        ix1 = jnp.maximum(px1, gx1)
        iy1 = jnp.maximum(py1, gy1)
        ix2 = jnp.minimum(px2, gx2)
        iy2 = jnp.minimum(py2, gy2)
        inter = jnp.maximum(ix2 - ix1, 0.0) * jnp.maximum(iy2 - iy1, 0.0)
        garea = (gx2 - gx1) * (gy2 - gy1)
        iou = inter / (parea + garea - inter + 1e-7)
        inside = ((ax >= gx1) & (ax <= gx2) & (ay >= gy1) & (ay <= gy2)
                  & valid)
        iou2 = iou * iou
        align = jnp.where(inside, sig * (iou2 * iou2 * iou2), -3.0)

        work = _key_from_align(align)
        cand_v = []
        cand_i = []
        for _ in range(TOPK):
            m = jnp.max(work, axis=0, keepdims=True)
            mb = jax.lax.bitcast_convert_type(m, jnp.int32)
            cand_v.append(m)
            cand_i.append((255 - (mb & 255)) * _L + liota1)
            work = jnp.where(work == m, jnp.float32(0.5), work)
        cand_v.extend([jnp.full((1, _L), 0.25, jnp.float32)] * 3)
        cand_i.extend([jnp.full((1, _L), BIGI, jnp.int32)] * 3)
        return (align, iou, jnp.concatenate(cand_v, axis=0),
                jnp.concatenate(cand_i, axis=0))

    def grp_step(grp, _):
        for j in range(8):
            al, io, cv, ci = _phase1(grp * 8 + j)
            alsc[j] = al
            iosc[j] = io
            cvsc[j * 16:(j + 1) * 16] = cv
            cisc[j * 16:(j + 1) * 16] = ci
        Wt = jnp.transpose(cvsc[...])
        It = jnp.transpose(cisc[...])
        mb = None
        ib = None
        for _ in range(TOPK):
            mb = _seg16_all(jnp.max(Wt, axis=0, keepdims=True), jnp.maximum)
            i1 = jnp.min(jnp.where(Wt == mb, It, BIGI), axis=0,
                         keepdims=True)
            ib = _seg16_all(i1, jnp.minimum)
            Wt = jnp.where(It == ib, jnp.float32(0.25), Wt)
        for j in range(8):
            g = grp * 8 + j
            t = _bcast_group(mb, j)
            ti = _bcast_group(ib, j)
            align = alsc[j]
            iou = iosc[j]
            key = _key_from_align(align)
            selmask = (key > t) | ((key == t) & (idx <= ti))
            upd = selmask & (align > metric[...])
            metric[...] = jnp.where(upd, align, metric[...])
            ovl[...] = jnp.where(upd, iou, ovl[...])
            tx1[...] = jnp.where(upd, gt_ref[0, g, 0], tx1[...])
            ty1[...] = jnp.where(upd, gt_ref[0, g, 1], ty1[...])
            tx2[...] = jnp.where(upd, gt_ref[0, g, 2], tx2[...])
            ty2[...] = jnp.where(upd, gt_ref[0, g, 3], ty2[...])
        return 0

    jax.lax.fori_loop(0, G // 8, grp_step, 0)

    fg = metric[...] > -0.5
    fgf = jnp.where(fg, 1.0, 0.0)
    npos = jnp.sum(fgf)
    denom = jnp.maximum(npos, 1.0)

    ts = jnp.where(fg, jnp.maximum(ovl[...], 0.1), 0.0)
    x = ps_ref[0]
    bce = (jnp.maximum(x, 0.0) - x * ts
           + jnp.log1p(jnp.exp(-jnp.abs(x))))
    match_b = jnp.sum(jnp.where(valid, bce, 0.0)) / nreal_f

    prob = sig
    pos_score_b = jnp.sum(jnp.where(fg, prob, 0.0))
    neg_score_b = jnp.sum(jnp.where(valid & (~fg), prob, 0.0))

    bx1 = jnp.where(fg, tx1[...], px1)
    by1 = jnp.where(fg, ty1[...], py1)
    bx2 = jnp.where(fg, tx2[...], px2)
    by2 = jnp.where(fg, ty2[...], py2)
    eps = 1e-7
    ix1 = jnp.maximum(px1, bx1)
    iy1 = jnp.maximum(py1, by1)
    ix2 = jnp.minimum(px2, bx2)
    iy2 = jnp.minimum(py2, by2)
    inter = jnp.maximum(ix2 - ix1, 0.0) * jnp.maximum(iy2 - iy1, 0.0)
    a2 = (bx2 - bx1) * (by2 - by1)
    iou = inter / (parea + a2 - inter + eps)
    matched_iou_b = jnp.sum(jnp.where(fg, iou, 0.0))
    cw = jnp.maximum(px2, bx2) - jnp.minimum(px1, bx1)
    ch = jnp.maximum(py2, by2) - jnp.minimum(py1, by1)
    c2 = cw * cw + ch * ch + eps
    rho2 = ((bx1 + bx2 - px1 - px2) ** 2 + (by1 + by2 - py1 - py2) ** 2) / 4.0
    w1 = px2 - px1 + eps
    h1 = py2 - py1 + eps
    w2 = bx2 - bx1 + eps
    h2 = by2 - by1 + eps
    pi2 = 9.869604401089358
    z = (w2 * h1 - w1 * h2) / (h1 * h2 + w1 * w2)
    az = jnp.abs(z)
    big = az > 1.0
    y = jnp.where(big, 1.0 / az, az)
    t = y * y
    p = jnp.float32(0.0028340642986113477)
    for coef in (-0.01600503050194432, 0.042587607462732255,
                 -0.0749544544309546, 0.10636754098013634,
                 -0.14202570511671397, 0.19992483578497475,
                 -0.3333306678069131, 0.9999999842426359):
        p = p * t + jnp.float32(coef)
    aty = y * p
    dang = jnp.sign(z) * jnp.where(big, 1.5707963267948966 - aty, aty)
    v = (4.0 / pi2) * dang * dang
    alpha = v / (v - iou + 1.0 + eps)
    ciou = iou - (rho2 / c2 + v * alpha)
    iou_b = jnp.sum(jnp.where(fg, 1.0 - ciou, 0.0)) / denom

    stv = st_ref[0]
    dsum = jnp.float32(0.0)
    tgt0 = (ax - bx1) / stv
    tgt1 = (ay - by1) / stv
    tgt2 = (bx2 - ax) / stv
    tgt3 = (by2 - ay) / stv
    for c, tgt in enumerate((tgt0, tgt1, tgt2, tgt3)):
        tgt = jnp.clip(tgt, 0.0, REG_MAX - 1 - 0.01)
        tl = tgt.astype(jnp.int32)
        tr = jnp.minimum(tl + 1, REG_MAX - 1)
        wl = tr.astype(jnp.float32) - tgt
        wr = 1.0 - wl
        lg = bd_ref[0, c * REG_MAX:(c + 1) * REG_MAX]
        m16 = jnp.max(lg, axis=0)
        s = jnp.sum(jnp.exp(lg - m16[None]), axis=0)
        lse = m16 + jnp.log(s)
        j3 = jax.lax.broadcasted_iota(jnp.int32, (REG_MAX, _R, _L), 0)
        l_tl = jnp.sum(jnp.where(tl[None] == j3, lg, 0.0), axis=0)
        l_tr = jnp.sum(jnp.where(tr[None] == j3, lg, 0.0), axis=0)
        dfl_c = (lse - l_tl) * wl + (lse - l_tr) * wr
        dsum = dsum + jnp.sum(jnp.where(fg, dfl_c, 0.0))
    dfl_b = dsum / (denom * 4.0)

    img = img_ref[0, 0].astype(jnp.float32)
    cx = (bx1 + bx2) * 0.5 / img
    cy = (by1 + by2) * 0.5 / img
    nrm = jnp.maximum(jnp.sqrt(cx * cx + cy * cy), 1e-12)
    pvx = pv_ref[0, 0, 0]
    pvy = pv_ref[0, 0, 1]
    pn = jnp.maximum(jnp.sqrt(pvx * pvx + pvy * pvy), 1e-12)
    contrast = 1.0 - (cx * (pvx / pn) + cy * (pvy / pn)) / nrm
    contrast_b = jnp.sum(jnp.where(fg, contrast, 0.0)) / denom

    out_ref[0, 0,0] = match_b
    out_ref[0, 0,1] = iou_b
    out_ref[0, 0,2] = dfl_b
    out_ref[0, 0,3] = contrast_b
    out_ref[0, 0,4] = npos
    out_ref[0, 0,5] = nreal_f - npos
    out_ref[0, 0,6] = pos_score_b
    out_ref[0, 0,7] = neg_score_b
    out_ref[0, 0,8] = matched_iou_b


def kernel(pred_boxes, pred_scores, anchor_points, stride_tensor,
           box_distribution, prompt_embedding, gt_boxes, image_size):
    B, N = pred_scores.shape
    G = gt_boxes.shape[1]
    pad = _NP - N

    ps = jnp.pad(pred_scores, ((0, 0), (0, pad))).reshape(B, _R, _L)
    pb = jnp.pad(jnp.transpose(pred_boxes, (0, 2, 1)),
                 ((0, 0), (0, 0), (0, pad))).reshape(B, 4, _R, _L)
    bd = jnp.pad(jnp.transpose(box_distribution, (0, 2, 3, 1)),
                 ((0, 0), (0, 0), (0, 0), (0, pad))
                 ).reshape(B, 4 * REG_MAX, _R, _L)
    an = jnp.pad(jnp.transpose(anchor_points, (1, 0)),
                 ((0, 0), (0, pad))).reshape(2, _R, _L)
    st = jnp.pad(stride_tensor, ((0, pad),),
                 constant_values=1.0).reshape(1, _R, _L)
    pv = prompt_embedding[:, :2].reshape(B, 1, 2)
    img = jnp.maximum(jnp.asarray(image_size, jnp.float32), 1.0)
    imgn = jnp.stack([img, jnp.float32(N)]).reshape(1, 2)

    out = pl.pallas_call(
        _loss_kernel,
        grid=(B,),
        in_specs=[
            pl.BlockSpec((1, _R, _L), lambda b: (b, 0, 0)),
            pl.BlockSpec((1, 4, _R, _L), lambda b: (b, 0, 0, 0)),
            pl.BlockSpec((1, 4 * REG_MAX, _R, _L), lambda b: (b, 0, 0, 0)),
            pl.BlockSpec((2, _R, _L), lambda b: (0, 0, 0)),
            pl.BlockSpec((1, _R, _L), lambda b: (0, 0, 0)),
            pl.BlockSpec((1, G, 4), lambda b: (b, 0, 0),
                         memory_space=pltpu.SMEM),
            pl.BlockSpec((1, 1, 2), lambda b: (b, 0, 0),
                         memory_space=pltpu.SMEM),
            pl.BlockSpec((1, 2), lambda b: (0, 0), memory_space=pltpu.SMEM),
        ],
        out_specs=pl.BlockSpec((1, 1, 16), lambda b: (b, 0, 0),
                               memory_space=pltpu.SMEM),
        out_shape=jax.ShapeDtypeStruct((B, 1, 16), jnp.float32),
        scratch_shapes=(
            [pltpu.VMEM((_R, _L), jnp.float32) for _ in range(6)]
            + [pltpu.VMEM((8, _R, _L), jnp.float32) for _ in range(2)]
            + [pltpu.VMEM((_L, _L), jnp.float32),
               pltpu.VMEM((_L, _L), jnp.int32)]),
        compiler_params=pltpu.CompilerParams(
            dimension_semantics=("parallel",)),
    )(ps, pb, bd, an, st, gt_boxes, pv, imgn)

    out = out[:, 0]
    match = out[:, 0]
    iou_l = out[:, 1]
    dfl_l = out[:, 2]
    contrast = out[:, 3]
    npos = out[:, 4]
    nneg = out[:, 5]
    pos_score = out[:, 6]
    neg_score = out[:, 7]
    matched_iou = out[:, 8]

    nb = float(B)
    total_match = jnp.sum(match)
    total_iou = jnp.sum(iou_l)
    total_dfl = jnp.sum(dfl_l)
    total_contrast = jnp.sum(contrast)
    total_pos = jnp.sum(npos)
    total_neg = jnp.sum(nneg)
    loss = (W_MATCH * total_match + W_IOU * total_iou + W_DFL * total_dfl
            + W_CONTRAST * total_contrast) / nb
    mean_pos_score = jnp.sum(pos_score) / jnp.maximum(total_pos, 1.0)
    mean_neg_score = jnp.sum(neg_score) / jnp.maximum(total_neg, 1.0)
    mean_matched_iou = jnp.sum(matched_iou) / jnp.maximum(total_pos, 1.0)
    return (loss, total_match / nb, total_iou / nb, total_dfl / nb,
            total_contrast / nb, total_pos, total_neg, mean_pos_score,
            mean_neg_score, mean_matched_iou)
